# Initial kernel scaffold; baseline (speedup 1.0000x reference)
import jax
import jax.numpy as jnp
from jax.experimental import pallas as pl


def kernel(userid, itemid, user_age, gender, user_occupation, item_kind, label, user_table, item_table, age_table, gender_table, occupation_table, kind_table):
    raise NotImplementedError("write your pallas kernel here")


# speedup vs baseline: 3.3825x; 3.3825x over previous
import functools

import jax
import jax.numpy as jnp
from jax import lax
from jax.experimental import pallas as pl
from jax.experimental.pallas import tpu as pltpu
from jax.experimental.pallas import tpu_sc as plsc

DIM = 16
B = 16384
K = 20
ROW = 1 + DIM

_NC = 2
_NS = 16
_NW = _NC * _NS
_BPW = B // _NW


def _sc_gather(user_table, item_table, uid, iid):
    mesh = plsc.VectorSubcoreMesh(core_axis_name="c", subcore_axis_name="s")

    @functools.partial(
        pl.kernel,
        mesh=mesh,
        compiler_params=pltpu.CompilerParams(use_tc_tiling_on_sc=False),
        out_type=(
            jax.ShapeDtypeStruct((B, ROW), jnp.float32),
            jax.ShapeDtypeStruct((B, ROW), jnp.float32),
        ),
        scratch_types=[
            pltpu.VMEM((_BPW,), jnp.int32),
            pltpu.VMEM((_BPW,), jnp.int32),
            pltpu.VMEM((_BPW, ROW), jnp.float32),
            pltpu.VMEM((_BPW, ROW), jnp.float32),
            pltpu.SemaphoreType.DMA,
            pltpu.SemaphoreType.DMA,
        ],
    )
    def k(utab, itab, uid_h, iid_h, urow_h, irow_h,
          uidx_v, iidx_v, urow_v, irow_v, usem, isem):
        wid = lax.axis_index("s") * _NC + lax.axis_index("c")
        base = wid * _BPW
        pltpu.sync_copy(uid_h.at[pl.ds(base, _BPW)], uidx_v)
        pltpu.sync_copy(iid_h.at[pl.ds(base, _BPW)], iidx_v)
        ucp = pltpu.async_copy(utab.at[uidx_v], urow_v, usem)
        icp = pltpu.async_copy(itab.at[iidx_v], irow_v, isem)
        ucp.wait()
        icp.wait()
        pltpu.sync_copy(urow_v, urow_h.at[pl.ds(base, _BPW)])
        pltpu.sync_copy(irow_v, irow_h.at[pl.ds(base, _BPW)])

    return k(user_table, item_table, uid, iid)


_BB = 2048


def _tc_body(urow_ref, irow_ref, age_ref, gen_ref, occ_ref, kind_ref, lab_ref,
             atab_ref, gtab_ref, otab_ref, ktab_ref, out_ref):
    f32 = jnp.float32
    u = urow_ref[...]
    i = irow_ref[...]
    s = u[:, 1:ROW] + i[:, 1:ROW]
    q = u[:, 1:ROW] * u[:, 1:ROW] + i[:, 1:ROW] * i[:, 1:ROW]
    bias = u[:, 0:1] + i[:, 0:1]

    def one_hot_feature(idx_col, tab, width):
        t = lax.broadcasted_iota(jnp.int32, (_BB, width), 1)
        oh = (idx_col == t).astype(f32)
        row = jnp.dot(oh, tab, precision=lax.Precision.HIGHEST,
                      preferred_element_type=f32)
        return row

    arow = one_hot_feature(age_ref[...], atab_ref[...], 8)
    grow = one_hot_feature(gen_ref[...], gtab_ref[...], 3)
    orow = one_hot_feature(occ_ref[...], otab_ref[...], 32)
    for row in (arow, grow, orow):
        bias = bias + row[:, 0:1]
        e = row[:, 1:ROW]
        s = s + e
        q = q + e * e

    kidx = kind_ref[...]
    t20 = lax.broadcasted_iota(jnp.int32, (_BB, K), 1)
    counts = jnp.zeros((_BB, K), f32)
    for k in range(K):
        counts = counts + (kidx[:, k:k + 1] == t20).astype(f32)
    counts = jnp.where(t20 != 0, counts, 0.0)
    ktab = ktab_ref[...]
    krow = jnp.dot(counts, ktab, precision=lax.Precision.HIGHEST,
                   preferred_element_type=f32)
    kemb2 = jnp.dot(counts, ktab[:, 1:ROW] * ktab[:, 1:ROW],
                    precision=lax.Precision.HIGHEST,
                    preferred_element_type=f32)
    bias = bias + krow[:, 0:1]
    s = s + krow[:, 1:ROW]
    q = q + kemb2

    two = 0.5 * (jnp.sum(s * s, axis=1, keepdims=True)
                 - jnp.sum(q, axis=1, keepdims=True))
    logit = bias + two
    p = 1.0 / (1.0 + jnp.exp(-logit))
    lab = lab_ref[...]
    bce = -(lab * jnp.log(p + 1e-6) + (1.0 - lab) * jnp.log(1.0 - p + 1e-6))
    part = jnp.sum(bce) * (1.0 / B)

    @pl.when(pl.program_id(0) == 0)
    def _():
        out_ref[...] = jnp.zeros_like(out_ref)

    out_ref[...] = out_ref[...] + part


def _tc_loss(urows, irows, age, gen, occ, kind, lab, atab, gtab, otab, ktab):
    grid = (B // _BB,)
    blk = lambda shape: pl.BlockSpec(shape, lambda i: (i, 0))
    rep = lambda shape: pl.BlockSpec(shape, lambda i: (0, 0))
    out = pl.pallas_call(
        _tc_body,
        grid=grid,
        in_specs=[
            blk((_BB, ROW)), blk((_BB, ROW)),
            blk((_BB, 1)), blk((_BB, 1)), blk((_BB, 1)),
            blk((_BB, K)), blk((_BB, 1)),
            rep((8, ROW)), rep((3, ROW)), rep((32, ROW)), rep((20, ROW)),
        ],
        out_specs=rep((1, 1)),
        out_shape=jax.ShapeDtypeStruct((1, 1), jnp.float32),
    )(urows, irows, age, gen, occ, kind, lab, atab, gtab, otab, ktab)
    return out[0, 0]


def kernel(userid, itemid, user_age, gender, user_occupation, item_kind,
           label, user_table, item_table, age_table, gender_table,
           occupation_table, kind_table):
    uid = userid.reshape(B).astype(jnp.int32)
    iid = itemid.reshape(B).astype(jnp.int32)
    urows, irows = _sc_gather(user_table, item_table, uid, iid)
    return _tc_loss(urows, irows, user_age, gender, user_occupation,
                    item_kind, label, age_table, gender_table,
                    occupation_table, kind_table)
